# Initial kernel scaffold; baseline (speedup 1.0000x reference)
import jax
import jax.numpy as jnp
from jax.experimental import pallas as pl


def kernel(x, W, b):
    raise NotImplementedError("write your pallas kernel here")


# speedup vs baseline: 2.2583x; 2.2583x over previous
import functools

import jax
import jax.numpy as jnp
from jax.experimental import pallas as pl

B, DIM, H, W_SP = 64, 768, 24, 24
NUM_EXPERTS = 16
HW = H * W_SP


def _router_body(x_ref, w_ref, b_ref, scores_ref, probs_ref, idx_ref):
    bb = x_ref.shape[0]
    for i in range(bb):
        logits = jnp.dot(w_ref[...], x_ref[i],
                         preferred_element_type=jnp.float32)
        logits = logits + b_ref[0, :][:, None]
        m = jnp.max(logits, axis=0, keepdims=True)
        e = jnp.exp(logits - m)
        s = jnp.sum(e, axis=0, keepdims=True)
        scores = e / s
        scores_ref[i] = scores

        lane = jax.lax.broadcasted_iota(jnp.int32, logits.shape, 0)
        i1 = jnp.argmax(logits, axis=0).astype(jnp.int32)
        masked = jnp.where(lane == i1[None, :], -jnp.inf, logits)
        i2 = jnp.argmax(masked, axis=0).astype(jnp.int32)
        v1 = jnp.max(scores, axis=0)
        v2 = jnp.max(jnp.where(lane == i1[None, :], -jnp.inf, scores), axis=0)
        t = v1 + v2
        probs_ref[i] = jnp.stack([v1 / t, v2 / t], axis=0)
        idx_ref[i] = jnp.stack([i1, i2], axis=0)


@functools.partial(jax.jit, static_argnames=())
def kernel(x, W, b):
    bsz = x.shape[0]
    xr = x.reshape(bsz, DIM, HW)
    bb = 1
    grid = (bsz // bb,)
    scores, probs, idx = pl.pallas_call(
        _router_body,
        grid=grid,
        in_specs=[
            pl.BlockSpec((bb, DIM, HW), lambda i: (i, 0, 0)),
            pl.BlockSpec((NUM_EXPERTS, DIM), lambda i: (0, 0)),
            pl.BlockSpec((1, NUM_EXPERTS), lambda i: (0, 0)),
        ],
        out_specs=[
            pl.BlockSpec((bb, NUM_EXPERTS, HW), lambda i: (i, 0, 0)),
            pl.BlockSpec((bb, 2, HW), lambda i: (i, 0, 0)),
            pl.BlockSpec((bb, 2, HW), lambda i: (i, 0, 0)),
        ],
        out_shape=[
            jax.ShapeDtypeStruct((bsz, NUM_EXPERTS, HW), jnp.float32),
            jax.ShapeDtypeStruct((bsz, 2, HW), jnp.float32),
            jax.ShapeDtypeStruct((bsz, 2, HW), jnp.int32),
        ],
    )(xr, W, b.reshape(1, NUM_EXPERTS))
    return (probs.reshape(bsz, 2, H, W_SP),
            idx.reshape(bsz, 2, H, W_SP),
            scores.reshape(bsz, NUM_EXPERTS, H, W_SP))


# speedup vs baseline: 2.6195x; 1.1599x over previous
import functools

import jax
import jax.numpy as jnp
from jax.experimental import pallas as pl
from jax.experimental.pallas import tpu as pltpu

B, DIM, H, W_SP = 64, 768, 24, 24
NUM_EXPERTS = 16
HW = H * W_SP


def _router_body(x_ref, w_ref, b_ref, scores_ref, probs_ref, idx_ref):
    bb = x_ref.shape[0]
    for i in range(bb):
        logits = jnp.dot(w_ref[...], x_ref[i],
                         preferred_element_type=jnp.float32)
        logits = logits + b_ref[0, :][:, None]
        m = jnp.max(logits, axis=0, keepdims=True)
        e = jnp.exp(logits - m)
        s = jnp.sum(e, axis=0, keepdims=True)
        scores = e / s
        scores_ref[i] = scores

        lane = jax.lax.broadcasted_iota(jnp.int32, logits.shape, 0)
        i1 = jnp.argmax(logits, axis=0).astype(jnp.int32)
        masked = jnp.where(lane == i1[None, :], -jnp.inf, logits)
        i2 = jnp.argmax(masked, axis=0).astype(jnp.int32)
        v1 = jnp.max(scores, axis=0)
        v2 = jnp.max(jnp.where(lane == i1[None, :], -jnp.inf, scores), axis=0)
        t = v1 + v2
        probs_ref[i] = jnp.stack([v1 / t, v2 / t], axis=0)
        idx_ref[i] = jnp.stack([i1, i2], axis=0)


@functools.partial(jax.jit, static_argnames=())
def kernel(x, W, b):
    bsz = x.shape[0]
    xr = x.reshape(bsz, DIM, HW)
    bb = 4
    grid = (bsz // bb,)
    scores, probs, idx = pl.pallas_call(
        _router_body,
        grid=grid,
        in_specs=[
            pl.BlockSpec((bb, DIM, HW), lambda i: (i, 0, 0)),
            pl.BlockSpec((NUM_EXPERTS, DIM), lambda i: (0, 0)),
            pl.BlockSpec((1, NUM_EXPERTS), lambda i: (0, 0)),
        ],
        out_specs=[
            pl.BlockSpec((bb, NUM_EXPERTS, HW), lambda i: (i, 0, 0)),
            pl.BlockSpec((bb, 2, HW), lambda i: (i, 0, 0)),
            pl.BlockSpec((bb, 2, HW), lambda i: (i, 0, 0)),
        ],
        out_shape=[
            jax.ShapeDtypeStruct((bsz, NUM_EXPERTS, HW), jnp.float32),
            jax.ShapeDtypeStruct((bsz, 2, HW), jnp.float32),
            jax.ShapeDtypeStruct((bsz, 2, HW), jnp.int32),
        ],
        compiler_params=pltpu.CompilerParams(
            dimension_semantics=("parallel",),
        ),
    )(xr, W, b.reshape(1, NUM_EXPERTS))
    return (probs.reshape(bsz, 2, H, W_SP),
            idx.reshape(bsz, 2, H, W_SP),
            scores.reshape(bsz, NUM_EXPERTS, H, W_SP))
